# baseline (speedup 1.0000x reference)
import jax
import jax.numpy as jnp
from jax import lax
from jax.experimental import pallas as pl
from jax.experimental.pallas import tpu as pltpu
from jax.experimental.pallas import tpu_sc as plsc

_B = 16384
_K = 64
_NC = 2
_NS = 16
_NW = _NC * _NS
_BPW = _B // _NW
_CH = 128
_NCH = _BPW // _CH
_L = 16


def _mf_body(uid_h, iid_h, uf_h, if_h, ub_h, ib_h, out_h,
             uidx, iidx, urows, irows, ubv, ibv, outv, sem):
    wid = lax.axis_index("s") * _NC + lax.axis_index("c")
    base = wid * _BPW

    pltpu.sync_copy(uid_h.at[wid], uidx)
    pltpu.sync_copy(iid_h.at[wid], iidx)

    copies = []
    for j in range(_NCH):
        r0 = j * _CH
        copies.append(pltpu.async_copy(uf_h.at[uidx.at[j]], urows.at[pl.ds(r0, _CH)], sem))
        copies.append(pltpu.async_copy(if_h.at[iidx.at[j]], irows.at[pl.ds(r0, _CH)], sem))
        copies.append(pltpu.async_copy(ub_h.at[uidx.at[j]], ubv.at[pl.ds(r0, _CH)], sem))
        copies.append(pltpu.async_copy(ib_h.at[iidx.at[j]], ibv.at[pl.ds(r0, _CH)], sem))
    for c in copies:
        c.wait()


    def group(g, carry):
        r0 = g * _L
        rows = r0 + lax.iota(jnp.int32, _L)
        acc = ubv[pl.ds(r0, _L)] + ibv[pl.ds(r0, _L)]
        for j in range(_K):
            cols = jnp.full((_L,), j, jnp.int32)
            acc = acc + (plsc.load_gather(urows, [rows, cols])
                         * plsc.load_gather(irows, [rows, cols]))
        outv[pl.ds(r0, _L)] = acc
        return carry

    lax.fori_loop(0, _BPW // _L, group, 0)
    pltpu.sync_copy(outv, out_h.at[pl.ds(base, _BPW)])


def kernel(user_id, item_id, user_factors, item_factors, user_bias, item_bias):
    uid = user_id.reshape(_NW, _NCH, _CH)
    iid = item_id.reshape(_NW, _NCH, _CH)
    ubf = user_bias.reshape(-1)
    ibf = item_bias.reshape(-1)
    mesh = plsc.VectorSubcoreMesh(core_axis_name="c", subcore_axis_name="s")
    f = pl.kernel(
        _mf_body,
        out_type=jax.ShapeDtypeStruct((_B,), jnp.float32),
        mesh=mesh,
        scratch_types=[
            pltpu.VMEM((_NCH, _CH), jnp.int32),
            pltpu.VMEM((_NCH, _CH), jnp.int32),
            pltpu.VMEM((_BPW, _K), jnp.float32),
            pltpu.VMEM((_BPW, _K), jnp.float32),
            pltpu.VMEM((_BPW,), jnp.float32),
            pltpu.VMEM((_BPW,), jnp.float32),
            pltpu.VMEM((_BPW,), jnp.float32),
            pltpu.SemaphoreType.DMA,
        ],
        compiler_params=pltpu.CompilerParams(needs_layout_passes=False, use_tc_tiling_on_sc=False),
    )
    return f(uid, iid, user_factors, item_factors, ubf, ibf)


# speedup vs baseline: 1.3963x; 1.3963x over previous
import jax
import jax.numpy as jnp
from jax import lax
from jax.experimental import pallas as pl
from jax.experimental.pallas import tpu as pltpu
from jax.experimental.pallas import tpu_sc as plsc

_B = 16384
_K = 64
_NC = 2
_NS = 16
_NW = _NC * _NS
_BPW = _B // _NW
_CH = 128
_NCH = _BPW // _CH
_L = 16
_G = 16
_NG = _BPW // _G


def _bias_body(uid_h, iid_h, ub_h, ib_h, out_h, uidx, iidx, ubg, ibg, sem):
    wid = lax.axis_index("s") * _NC + lax.axis_index("c")
    base = wid * _BPW
    for c in range(_NCH):
        pltpu.sync_copy(uid_h.at[pl.ds(base + c * _CH, _CH)], uidx.at[c])
        pltpu.sync_copy(iid_h.at[pl.ds(base + c * _CH, _CH)], iidx.at[c])
    cps = []
    for c in range(_NCH):
        cps.append(pltpu.async_copy(ub_h.at[uidx.at[c]], ubg.at[pl.ds(c * _CH, _CH)], sem))
        cps.append(pltpu.async_copy(ib_h.at[iidx.at[c]], ibg.at[pl.ds(c * _CH, _CH)], sem))
    for cp in cps:
        cp.wait()

    def body(i, carry):
        ubg[pl.ds(i * _L, _L)] = ubg[pl.ds(i * _L, _L)] + ibg[pl.ds(i * _L, _L)]
        return carry

    lax.fori_loop(0, _BPW // _L, body, 0)
    pltpu.sync_copy(ubg, out_h.at[pl.ds(base, _BPW)])


def _dot_body(uid_h, iid_h, uf_h, if_h, bs_h, out_h,
              uidx, iidx, bsv, ru0, ru1, ri0, ri1, outv, semu, semi):
    wid = lax.axis_index("s") * _NC + lax.axis_index("c")
    base = wid * _BPW
    pltpu.sync_copy(uid_h.at[pl.ds(base, _BPW)], uidx)
    pltpu.sync_copy(iid_h.at[pl.ds(base, _BPW)], iidx)
    pltpu.sync_copy(bs_h.at[pl.ds(base, _BPW)], bsv)

    def fire(g, ru, ri):
        r0 = g * _G
        uvec = uidx[pl.ds(r0, _G)]
        ivec = iidx[pl.ds(r0, _G)]
        for l in range(_G):
            pltpu.async_copy(uf_h.at[uvec[l]], ru.at[l], semu)
            pltpu.async_copy(if_h.at[ivec[l]], ri.at[l], semi)

    def drain(ru, ri):
        pltpu.make_async_copy(uf_h.at[pl.ds(0, _G)], ru, semu).wait()
        pltpu.make_async_copy(if_h.at[pl.ds(0, _G)], ri, semi).wait()

    lanes = lax.iota(jnp.int32, _L)

    def compute(g, ru, ri):
        acc = bsv[pl.ds(g * _G, _G)]
        for j in range(_K):
            cols = jnp.full((_L,), j, jnp.int32)
            acc = acc + (plsc.load_gather(ru, [lanes, cols])
                         * plsc.load_gather(ri, [lanes, cols]))
        outv[pl.ds(g * _G, _G)] = acc

    fire(0, ru0, ri0)

    def pair(h, carry):
        g0 = 2 * h
        g1 = g0 + 1

        @pl.when(g1 < _NG)
        def _():
            fire(g1, ru1, ri1)

        drain(ru0, ri0)
        compute(g0, ru0, ri0)

        @pl.when(g0 + 2 < _NG)
        def _():
            fire(g0 + 2, ru0, ri0)

        @pl.when(g1 < _NG)
        def _():
            drain(ru1, ri1)
            compute(g1, ru1, ri1)

        return carry

    lax.fori_loop(0, (_NG + 1) // 2, pair, 0)
    pltpu.sync_copy(outv, out_h.at[pl.ds(base, _BPW)])


def kernel(user_id, item_id, user_factors, item_factors, user_bias, item_bias):
    uid = user_id.reshape(_B)
    iid = item_id.reshape(_B)
    ubf = user_bias.reshape(user_bias.shape[0])
    ibf = item_bias.reshape(item_bias.shape[0])
    mesh = plsc.VectorSubcoreMesh(core_axis_name="c", subcore_axis_name="s")

    bias_call = pl.kernel(
        _bias_body,
        out_type=jax.ShapeDtypeStruct((_B,), jnp.float32),
        mesh=mesh,
        scratch_types=[
            pltpu.VMEM((_NCH, _CH), jnp.int32),
            pltpu.VMEM((_NCH, _CH), jnp.int32),
            pltpu.VMEM((_BPW,), jnp.float32),
            pltpu.VMEM((_BPW,), jnp.float32),
            pltpu.SemaphoreType.DMA,
        ],
        compiler_params=pltpu.CompilerParams(
            needs_layout_passes=False, use_tc_tiling_on_sc=False),
    )
    bias_sum = bias_call(uid, iid, ubf, ibf)

    dot_call = pl.kernel(
        _dot_body,
        out_type=jax.ShapeDtypeStruct((_B,), jnp.float32),
        mesh=mesh,
        scratch_types=[
            pltpu.VMEM((_BPW,), jnp.int32),
            pltpu.VMEM((_BPW,), jnp.int32),
            pltpu.VMEM((_BPW,), jnp.float32),
            pltpu.VMEM((_G, _K), jnp.float32),
            pltpu.VMEM((_G, _K), jnp.float32),
            pltpu.VMEM((_G, _K), jnp.float32),
            pltpu.VMEM((_G, _K), jnp.float32),
            pltpu.VMEM((_BPW,), jnp.float32),
            pltpu.SemaphoreType.DMA,
            pltpu.SemaphoreType.DMA,
        ],
        compiler_params=pltpu.CompilerParams(
            needs_layout_passes=False, use_tc_tiling_on_sc=True),
    )
    return dot_call(uid, iid, user_factors, item_factors, bias_sum)
